# baseline (speedup 1.0000x reference)
import functools

import jax
import jax.numpy as jnp
from jax import lax
from jax.experimental import pallas as pl
from jax.experimental.pallas import tpu as pltpu
from jax.experimental.pallas import tpu_sc as plsc

TOKENS = 8192
DIM = 4096
NUM_EXPERTS = 64
BT = 512
NB = TOKENS // BT
NW = 32
TPW = TOKENS // NW
CHUNK = 16
NCHUNK = TPW // CHUNK

_HI = jax.lax.Precision.HIGHEST


def _router_block(x_ref, w_ref, p_ref, counts_ref,
                  meta_s, probs_s, offs_s, carry_ref):
    b = pl.program_id(0)

    @pl.when(b == 0)
    def _init():
        carry_ref[...] = jnp.zeros_like(carry_ref)

    ecol = lax.broadcasted_iota(jnp.int32, (NUM_EXPERTS, 1), 0)
    pow2 = lax.bitcast_convert_type((127 - 2 * ecol) << 23, jnp.float32)
    r2 = lax.broadcasted_iota(jnp.int32, (2 * NUM_EXPERTS, 2), 0)
    c2 = lax.broadcasted_iota(jnp.int32, (2 * NUM_EXPERTS, 2), 1)
    bdiag = (jnp.right_shift(r2, 6) == c2).astype(jnp.float32)

    def first_hit(oh):
        v = lax.dot_general(oh, pow2, (((1,), (0,)), ((), ())),
                            preferred_element_type=jnp.float32)
        bits = lax.bitcast_convert_type(v, jnp.int32)
        expf = jnp.bitwise_and(jnp.right_shift(bits, 23), 255)
        return jnp.right_shift(127 - expf, 1)

    def pair_sum(a0, a1):
        cat = jnp.concatenate([a0, a1], axis=1)
        return lax.dot_general(cat, bdiag, (((1,), (0,)), ((), ())),
                               preferred_element_type=jnp.float32,
                               precision=_HI)

    @pl.when(b < NB)
    def _phase_a():
        x = x_ref[...]
        w = w_ref[...]
        scores = lax.dot_general(x, w, (((1,), (1,)), ((), ())),
                                 preferred_element_type=jnp.float32)
        col = lax.broadcasted_iota(jnp.int32, (BT, NUM_EXPERTS), 1)
        m1 = jnp.max(scores, axis=1, keepdims=True)
        oh1 = (scores == m1).astype(jnp.float32)
        idx1 = first_hit(oh1)
        is1 = col == idx1
        masked = jnp.where(is1, -jnp.inf, scores)
        m2 = jnp.max(masked, axis=1, keepdims=True)
        idx2 = first_hit((masked == m2).astype(jnp.float32))
        is2 = col == idx2
        z = jnp.exp(m2 - m1)
        q1 = 1.0 / (1.0 + z)
        q2 = z / (1.0 + z)
        oh1e = is1.astype(jnp.float32)
        oh2e = is2.astype(jnp.float32)
        combined = oh1e + oh2e
        rowi = lax.broadcasted_iota(jnp.int32, (BT, BT), 0)
        coli = lax.broadcasted_iota(jnp.int32, (BT, BT), 1)
        tri = (rowi >= coli).astype(jnp.float32)
        incl = lax.dot_general(tri, combined, (((1,), (0,)), ((), ())),
                               preferred_element_type=jnp.float32)
        excl = incl - combined
        base = carry_ref[0:1, 0:NUM_EXPERTS]
        cnt = excl + base
        r01 = pair_sum(oh1e * cnt, oh2e * cnt)
        carry_ref[0:1, 0:NUM_EXPERTS] = base + incl[BT - 1:BT, :]
        meta_s[pl.ds(b * BT, BT), :] = jnp.concatenate(
            [idx1, idx2, r01.astype(jnp.int32),
             jnp.zeros((BT, 4), jnp.int32)], axis=1)
        probs_s[pl.ds(b * BT, BT), :] = jnp.concatenate(
            [q1, q2, jnp.zeros((BT, 6), jnp.float32)], axis=1)

        @pl.when(b == NB - 1)
        def _fin():
            counts_ref[...] = carry_ref[...]

    @pl.when(b >= NB)
    def _phase_b():
        @pl.when(b == NB)
        def _mkoffs():
            r128 = lax.broadcasted_iota(jnp.int32, (128, 128), 0)
            c128 = lax.broadcasted_iota(jnp.int32, (128, 128), 1)
            below = (r128 < c128).astype(jnp.float32)
            offs_s[...] = lax.dot_general(carry_ref[...], below,
                                          (((1,), (0,)), ((), ())),
                                          preferred_element_type=jnp.float32,
                                          precision=_HI)

        bb = b - NB
        ms = meta_s[pl.ds(bb * BT, BT), :]
        qs = probs_s[pl.ds(bb * BT, BT), :]
        col = lax.broadcasted_iota(jnp.int32, (BT, NUM_EXPERTS), 1)
        offs = offs_s[0:1, 0:NUM_EXPERTS]
        oh1b = (col == ms[:, 0:1]).astype(jnp.float32)
        oh2b = (col == ms[:, 1:2]).astype(jnp.float32)
        o01 = pair_sum(oh1b * offs, oh2b * offs)
        p0 = o01[:, 0:1] + ms[:, 2:3].astype(jnp.float32)
        p1 = o01[:, 1:2] + ms[:, 3:4].astype(jnp.float32)
        rowi = lax.broadcasted_iota(jnp.int32, (BT, BT), 0)
        coli = lax.broadcasted_iota(jnp.int32, (BT, BT), 1)
        eye = (rowi == coli).astype(jnp.float32)
        pt = lax.dot_general(jnp.concatenate([p0, p1], axis=1), eye,
                             (((0,), (0,)), ((), ())),
                             preferred_element_type=jnp.float32,
                             precision=_HI)
        qt = lax.dot_general(qs[:, 0:2], eye, (((0,), (0,)), ((), ())),
                             preferred_element_type=jnp.float32,
                             precision=_HI)
        p_ref[...] = jnp.concatenate(
            [pt.astype(jnp.int32),
             lax.bitcast_convert_type(qt, jnp.int32),
             jnp.zeros((4, BT), jnp.int32)], axis=0)


_router = pl.pallas_call(
    _router_block,
    grid=(2 * NB,),
    in_specs=[
        pl.BlockSpec((BT, DIM), lambda b: (jnp.minimum(b, NB - 1), 0)),
        pl.BlockSpec((NUM_EXPERTS, DIM), lambda b: (0, 0)),
    ],
    out_specs=[
        pl.BlockSpec((8, BT), lambda b: (0, jnp.maximum(b - NB, 0))),
        pl.BlockSpec((8, 128), lambda b: (0, 0)),
    ],
    out_shape=[
        jax.ShapeDtypeStruct((8, TOKENS), jnp.int32),
        jax.ShapeDtypeStruct((8, 128), jnp.float32),
    ],
    scratch_shapes=[
        pltpu.VMEM((TOKENS, 8), jnp.int32),
        pltpu.VMEM((TOKENS, 8), jnp.float32),
        pltpu.VMEM((8, 128), jnp.float32),
        pltpu.VMEM((8, 128), jnp.float32),
    ],
)


HDIM = DIM // 2


def _dispatch_body(x_hbm, pqi_hbm, pqf_hbm, xg_hbm, sc_hbm, ss_hbm,
                   pqi_v, pqf_v, buf0, buf1, d00, d01, d10, d11,
                   p0w, p1w, vtw, q0w, q1w,
                   sem_l0, sem_l1, sem_s0, sem_s1, sem_w0, sem_w1,
                   sem_p0, sem_p1):
    cid = lax.axis_index("c")
    sid = lax.axis_index("s")
    wid = sid * 2 + cid
    base_tok = wid * TPW
    iota = lax.iota(jnp.int32, 16)

    def drain(src, dst, sem):
        pltpu.make_async_copy(src, dst, sem).wait()

    def load(tok, h, buf, sem):
        pltpu.async_copy(
            x_hbm.at[pl.ds(tok, CHUNK), pl.ds(h * HDIM, HDIM)], buf, sem)

    def drain_load(tok, h, buf, sem):
        pltpu.make_async_copy(
            x_hbm.at[pl.ds(tok, CHUNK), pl.ds(h * HDIM, HDIM)], buf,
            sem).wait()

    def pq_start(c, wi, sem):
        pltpu.async_copy(pqi_hbm.at[wid * NCHUNK + c], pqi_v.at[wi], sem)
        pltpu.async_copy(pqf_hbm.at[wid * NCHUNK + c], pqf_v.at[wi], sem)

    def pq_drain(c, wi, sem):
        pltpu.make_async_copy(
            pqi_hbm.at[wid * NCHUNK + c], pqi_v.at[wi], sem).wait()
        pltpu.make_async_copy(
            pqf_hbm.at[wid * NCHUNK + c], pqf_v.at[wi], sem).wait()

    load(base_tok, 0, buf0, sem_l0)
    pq_start(0, 0, sem_p0)

    @pl.loop(0, NCHUNK, step=2)
    def _outer(cbase):
        for cc in (0, 1):
            c = cbase + cc
            tok = base_tok + c * CHUNK
            wi = cc
            sem_w = sem_w0 if cc == 0 else sem_w1
            sem_p = sem_p0 if cc == 0 else sem_p1
            sem_pn = sem_p1 if cc == 0 else sem_p0
            pq_drain(c, wi, sem_p)
            @pl.when(c < NCHUNK - 1)
            def _pq_next():
                pq_start(c + 1, 1 - wi, sem_pn)
            p0v = pqi_v[wi, 0]
            p1v = pqi_v[wi, 1]
            drain_load(tok, 0, buf0, sem_l0)
            d00[...] = p0v
            d01[...] = p1v
            pltpu.async_copy(buf0, xg_hbm.at[d00, pl.ds(0, HDIM)], sem_s0)
            pltpu.async_copy(buf0, xg_hbm.at[d01, pl.ds(0, HDIM)], sem_s0)
            @pl.when(c >= 2)
            def _drain_words():
                drain(vtw.at[wi], sc_hbm.at[p0w.at[wi]], sem_w)
                drain(vtw.at[wi], sc_hbm.at[p1w.at[wi]], sem_w)
                drain(q0w.at[wi], ss_hbm.at[p0w.at[wi]], sem_w)
                drain(q1w.at[wi], ss_hbm.at[p1w.at[wi]], sem_w)
            p0w[wi, :] = p0v
            p1w[wi, :] = p1v
            vtw[wi, :] = tok + iota
            q0w[wi, :] = pqf_v[wi, 0]
            q1w[wi, :] = pqf_v[wi, 1]
            pltpu.async_copy(vtw.at[wi], sc_hbm.at[p0w.at[wi]], sem_w)
            pltpu.async_copy(vtw.at[wi], sc_hbm.at[p1w.at[wi]], sem_w)
            pltpu.async_copy(q0w.at[wi], ss_hbm.at[p0w.at[wi]], sem_w)
            pltpu.async_copy(q1w.at[wi], ss_hbm.at[p1w.at[wi]], sem_w)
            @pl.when(c >= 1)
            def _drain_s1():
                drain(buf1, xg_hbm.at[d10, pl.ds(HDIM, HDIM)], sem_s1)
                drain(buf1, xg_hbm.at[d11, pl.ds(HDIM, HDIM)], sem_s1)
            load(tok, 1, buf1, sem_l1)
            drain_load(tok, 1, buf1, sem_l1)
            d10[...] = p0v
            d11[...] = p1v
            pltpu.async_copy(buf1, xg_hbm.at[d10, pl.ds(HDIM, HDIM)], sem_s1)
            pltpu.async_copy(buf1, xg_hbm.at[d11, pl.ds(HDIM, HDIM)], sem_s1)
            drain(buf0, xg_hbm.at[d00, pl.ds(0, HDIM)], sem_s0)
            drain(buf0, xg_hbm.at[d01, pl.ds(0, HDIM)], sem_s0)
            @pl.when(c < NCHUNK - 1)
            def _next_load():
                load(tok + CHUNK, 0, buf0, sem_l0)

    drain(buf1, xg_hbm.at[d10, pl.ds(HDIM, HDIM)], sem_s1)
    drain(buf1, xg_hbm.at[d11, pl.ds(HDIM, HDIM)], sem_s1)
    for wi, sem_w in ((0, sem_w0), (1, sem_w1)):
        drain(vtw.at[wi], sc_hbm.at[p0w.at[wi]], sem_w)
        drain(vtw.at[wi], sc_hbm.at[p1w.at[wi]], sem_w)
        drain(q0w.at[wi], ss_hbm.at[p0w.at[wi]], sem_w)
        drain(q1w.at[wi], ss_hbm.at[p1w.at[wi]], sem_w)


@functools.cache
def _make_dispatch():
    return functools.partial(
        pl.kernel,
        out_type=[
            jax.ShapeDtypeStruct((2 * TOKENS, DIM), jnp.float32),
            jax.ShapeDtypeStruct((2 * TOKENS,), jnp.int32),
            jax.ShapeDtypeStruct((2 * TOKENS,), jnp.float32),
        ],
        mesh=plsc.VectorSubcoreMesh(core_axis_name="c", subcore_axis_name="s",
                                    num_cores=2, num_subcores=16),
        scratch_types=[
            pltpu.VMEM((2, 2, 16), jnp.int32),
            pltpu.VMEM((2, 2, 16), jnp.float32),
            pltpu.VMEM((CHUNK, HDIM), jnp.float32),
            pltpu.VMEM((CHUNK, HDIM), jnp.float32),
            pltpu.VMEM((16,), jnp.int32),
            pltpu.VMEM((16,), jnp.int32),
            pltpu.VMEM((16,), jnp.int32),
            pltpu.VMEM((16,), jnp.int32),
            pltpu.VMEM((2, 16), jnp.int32),
            pltpu.VMEM((2, 16), jnp.int32),
            pltpu.VMEM((2, 16), jnp.int32),
            pltpu.VMEM((2, 16), jnp.float32),
            pltpu.VMEM((2, 16), jnp.float32),
            pltpu.SemaphoreType.DMA,
            pltpu.SemaphoreType.DMA,
            pltpu.SemaphoreType.DMA,
            pltpu.SemaphoreType.DMA,
            pltpu.SemaphoreType.DMA,
            pltpu.SemaphoreType.DMA,
            pltpu.SemaphoreType.DMA,
            pltpu.SemaphoreType.DMA,
        ],
    )(_dispatch_body)


def kernel(x, W):
    p, counts = _router(x, W)
    pqi = p[:2].reshape(2, TOKENS // CHUNK, CHUNK).transpose(1, 0, 2)
    pqf = lax.bitcast_convert_type(
        p[2:4], jnp.float32).reshape(2, TOKENS // CHUNK, CHUNK).transpose(1, 0, 2)
    x_gathered, scatter_indices, scores_sorted = _make_dispatch()(x, pqi, pqf)
    num_tokens_per_expert = counts[0, :NUM_EXPERTS]
    return (x_gathered, num_tokens_per_expert, scatter_indices, scores_sorted)


# speedup vs baseline: 1.0326x; 1.0326x over previous
import functools

import jax
import jax.numpy as jnp
from jax import lax
from jax.experimental import pallas as pl
from jax.experimental.pallas import tpu as pltpu
from jax.experimental.pallas import tpu_sc as plsc

TOKENS = 8192
DIM = 4096
NUM_EXPERTS = 64
BT = 512
NB = TOKENS // BT
NW = 32
TPW = TOKENS // NW
CHUNK = 16
NCHUNK = TPW // CHUNK

_HI = jax.lax.Precision.HIGHEST


def _router_block(x_ref, w_ref, p_ref, counts_ref,
                  meta_s, probs_s, offs_s, carry_ref):
    b = pl.program_id(0)

    @pl.when(b == 0)
    def _init():
        carry_ref[...] = jnp.zeros_like(carry_ref)

    @pl.when(b < NB)
    def _phase_a():
        x = x_ref[...]
        w = w_ref[...]
        scores = lax.dot_general(x, w, (((1,), (1,)), ((), ())),
                                 preferred_element_type=jnp.float32)
        col = lax.broadcasted_iota(jnp.int32, (BT, NUM_EXPERTS), 1)
        m1 = jnp.max(scores, axis=1, keepdims=True)
        idx1 = jnp.min(jnp.where(scores == m1, col, NUM_EXPERTS), axis=1,
                       keepdims=True)
        is1 = col == idx1
        masked = jnp.where(is1, -jnp.inf, scores)
        m2 = jnp.max(masked, axis=1, keepdims=True)
        idx2 = jnp.min(jnp.where(masked == m2, col, NUM_EXPERTS), axis=1,
                       keepdims=True)
        is2 = col == idx2
        z = jnp.exp(m2 - m1)
        q1 = 1.0 / (1.0 + z)
        q2 = z / (1.0 + z)
        combined = is1.astype(jnp.float32) + is2.astype(jnp.float32)
        rowi = lax.broadcasted_iota(jnp.int32, (BT, BT), 0)
        coli = lax.broadcasted_iota(jnp.int32, (BT, BT), 1)
        tri = (rowi >= coli).astype(jnp.float32)
        incl = lax.dot_general(tri, combined, (((1,), (0,)), ((), ())),
                               preferred_element_type=jnp.float32)
        excl = incl - combined
        base = carry_ref[0:1, 0:NUM_EXPERTS]
        cnt = excl + base
        r0 = jnp.sum(jnp.where(is1, cnt, 0.0), axis=1, keepdims=True)
        r1 = jnp.sum(jnp.where(is2, cnt, 0.0), axis=1, keepdims=True)
        carry_ref[0:1, 0:NUM_EXPERTS] = base + incl[BT - 1:BT, :]
        zi = jnp.zeros((BT, 124), jnp.int32)
        meta_s[pl.ds(b * BT, BT), :] = jnp.concatenate(
            [idx1, idx2, r0.astype(jnp.int32), r1.astype(jnp.int32), zi],
            axis=1)
        zf = jnp.zeros((BT, 126), jnp.float32)
        probs_s[pl.ds(b * BT, BT), :] = jnp.concatenate([q1, q2, zf], axis=1)

        @pl.when(b == NB - 1)
        def _fin():
            counts_ref[...] = carry_ref[...]

    @pl.when(b >= NB)
    def _phase_b():
        @pl.when(b == NB)
        def _mkoffs():
            r128 = lax.broadcasted_iota(jnp.int32, (128, 128), 0)
            c128 = lax.broadcasted_iota(jnp.int32, (128, 128), 1)
            below = (r128 < c128).astype(jnp.float32)
            offs_s[...] = lax.dot_general(carry_ref[...], below,
                                          (((1,), (0,)), ((), ())),
                                          preferred_element_type=jnp.float32,
                                          precision=_HI)

        bb = b - NB
        ms = meta_s[pl.ds(bb * BT, BT), :]
        qs = probs_s[pl.ds(bb * BT, BT), :]
        col = lax.broadcasted_iota(jnp.int32, (BT, NUM_EXPERTS), 1)
        offs = offs_s[0:1, 0:NUM_EXPERTS]
        zero = jnp.zeros((BT, NUM_EXPERTS), jnp.float32)
        o0 = jnp.sum(jnp.where(col == ms[:, 0:1], offs + zero, 0.0),
                     axis=1, keepdims=True)
        o1 = jnp.sum(jnp.where(col == ms[:, 1:2], offs + zero, 0.0),
                     axis=1, keepdims=True)
        p0 = o0 + ms[:, 2:3].astype(jnp.float32)
        p1 = o1 + ms[:, 3:4].astype(jnp.float32)
        rowi = lax.broadcasted_iota(jnp.int32, (BT, BT), 0)
        coli = lax.broadcasted_iota(jnp.int32, (BT, BT), 1)
        eye = (rowi == coli).astype(jnp.float32)
        pt = lax.dot_general(jnp.concatenate([p0, p1], axis=1), eye,
                             (((0,), (0,)), ((), ())),
                             preferred_element_type=jnp.float32,
                             precision=_HI)
        qt = lax.dot_general(qs[:, 0:2], eye, (((0,), (0,)), ((), ())),
                             preferred_element_type=jnp.float32,
                             precision=_HI)
        p_ref[...] = jnp.concatenate(
            [pt.astype(jnp.int32),
             lax.bitcast_convert_type(qt, jnp.int32),
             jnp.zeros((4, BT), jnp.int32)], axis=0)


_router = pl.pallas_call(
    _router_block,
    grid=(2 * NB,),
    in_specs=[
        pl.BlockSpec((BT, DIM), lambda b: (jnp.minimum(b, NB - 1), 0)),
        pl.BlockSpec((NUM_EXPERTS, DIM), lambda b: (0, 0)),
    ],
    out_specs=[
        pl.BlockSpec((8, BT), lambda b: (0, jnp.maximum(b - NB, 0))),
        pl.BlockSpec((8, 128), lambda b: (0, 0)),
    ],
    out_shape=[
        jax.ShapeDtypeStruct((8, TOKENS), jnp.int32),
        jax.ShapeDtypeStruct((8, 128), jnp.float32),
    ],
    scratch_shapes=[
        pltpu.VMEM((TOKENS, 128), jnp.int32),
        pltpu.VMEM((TOKENS, 128), jnp.float32),
        pltpu.VMEM((8, 128), jnp.float32),
        pltpu.VMEM((8, 128), jnp.float32),
    ],
)


HDIM = DIM // 2


def _dispatch_body(x_hbm, pqi_hbm, pqf_hbm, xg_hbm, sc_hbm, ss_hbm,
                   pqi_v, pqf_v, buf0, buf1, d00, d01, d10, d11,
                   p0w, p1w, vtw, q0w, q1w,
                   sem_l0, sem_l1, sem_s0, sem_s1, sem_w0, sem_w1,
                   sem_p0, sem_p1):
    cid = lax.axis_index("c")
    sid = lax.axis_index("s")
    wid = sid * 2 + cid
    base_tok = wid * TPW
    iota = lax.iota(jnp.int32, 16)

    def drain(src, dst, sem):
        pltpu.make_async_copy(src, dst, sem).wait()

    def load(tok, h, buf, sem):
        pltpu.async_copy(
            x_hbm.at[pl.ds(tok, CHUNK), pl.ds(h * HDIM, HDIM)], buf, sem)

    def drain_load(tok, h, buf, sem):
        pltpu.make_async_copy(
            x_hbm.at[pl.ds(tok, CHUNK), pl.ds(h * HDIM, HDIM)], buf,
            sem).wait()

    def pq_start(c, wi, sem):
        pltpu.async_copy(pqi_hbm.at[wid * NCHUNK + c], pqi_v.at[wi], sem)
        pltpu.async_copy(pqf_hbm.at[wid * NCHUNK + c], pqf_v.at[wi], sem)

    def pq_drain(c, wi, sem):
        pltpu.make_async_copy(
            pqi_hbm.at[wid * NCHUNK + c], pqi_v.at[wi], sem).wait()
        pltpu.make_async_copy(
            pqf_hbm.at[wid * NCHUNK + c], pqf_v.at[wi], sem).wait()

    load(base_tok, 0, buf0, sem_l0)
    pq_start(0, 0, sem_p0)

    @pl.loop(0, NCHUNK, step=2)
    def _outer(cbase):
        for cc in (0, 1):
            c = cbase + cc
            tok = base_tok + c * CHUNK
            wi = cc
            sem_w = sem_w0 if cc == 0 else sem_w1
            sem_p = sem_p0 if cc == 0 else sem_p1
            sem_pn = sem_p1 if cc == 0 else sem_p0
            pq_drain(c, wi, sem_p)
            @pl.when(c < NCHUNK - 1)
            def _pq_next():
                pq_start(c + 1, 1 - wi, sem_pn)
            p0v = pqi_v[wi, 0]
            p1v = pqi_v[wi, 1]
            drain_load(tok, 0, buf0, sem_l0)
            d00[...] = p0v
            d01[...] = p1v
            pltpu.async_copy(buf0, xg_hbm.at[d00, pl.ds(0, HDIM)], sem_s0)
            pltpu.async_copy(buf0, xg_hbm.at[d01, pl.ds(0, HDIM)], sem_s0)
            @pl.when(c >= 2)
            def _drain_words():
                drain(vtw.at[wi], sc_hbm.at[p0w.at[wi]], sem_w)
                drain(vtw.at[wi], sc_hbm.at[p1w.at[wi]], sem_w)
                drain(q0w.at[wi], ss_hbm.at[p0w.at[wi]], sem_w)
                drain(q1w.at[wi], ss_hbm.at[p1w.at[wi]], sem_w)
            p0w[wi, :] = p0v
            p1w[wi, :] = p1v
            vtw[wi, :] = tok + iota
            q0w[wi, :] = pqf_v[wi, 0]
            q1w[wi, :] = pqf_v[wi, 1]
            pltpu.async_copy(vtw.at[wi], sc_hbm.at[p0w.at[wi]], sem_w)
            pltpu.async_copy(vtw.at[wi], sc_hbm.at[p1w.at[wi]], sem_w)
            pltpu.async_copy(q0w.at[wi], ss_hbm.at[p0w.at[wi]], sem_w)
            pltpu.async_copy(q1w.at[wi], ss_hbm.at[p1w.at[wi]], sem_w)
            @pl.when(c >= 1)
            def _drain_s1():
                drain(buf1, xg_hbm.at[d10, pl.ds(HDIM, HDIM)], sem_s1)
                drain(buf1, xg_hbm.at[d11, pl.ds(HDIM, HDIM)], sem_s1)
            load(tok, 1, buf1, sem_l1)
            drain_load(tok, 1, buf1, sem_l1)
            d10[...] = p0v
            d11[...] = p1v
            pltpu.async_copy(buf1, xg_hbm.at[d10, pl.ds(HDIM, HDIM)], sem_s1)
            pltpu.async_copy(buf1, xg_hbm.at[d11, pl.ds(HDIM, HDIM)], sem_s1)
            drain(buf0, xg_hbm.at[d00, pl.ds(0, HDIM)], sem_s0)
            drain(buf0, xg_hbm.at[d01, pl.ds(0, HDIM)], sem_s0)
            @pl.when(c < NCHUNK - 1)
            def _next_load():
                load(tok + CHUNK, 0, buf0, sem_l0)

    drain(buf1, xg_hbm.at[d10, pl.ds(HDIM, HDIM)], sem_s1)
    drain(buf1, xg_hbm.at[d11, pl.ds(HDIM, HDIM)], sem_s1)
    for wi, sem_w in ((0, sem_w0), (1, sem_w1)):
        drain(vtw.at[wi], sc_hbm.at[p0w.at[wi]], sem_w)
        drain(vtw.at[wi], sc_hbm.at[p1w.at[wi]], sem_w)
        drain(q0w.at[wi], ss_hbm.at[p0w.at[wi]], sem_w)
        drain(q1w.at[wi], ss_hbm.at[p1w.at[wi]], sem_w)


@functools.cache
def _make_dispatch():
    return functools.partial(
        pl.kernel,
        out_type=[
            jax.ShapeDtypeStruct((2 * TOKENS, DIM), jnp.float32),
            jax.ShapeDtypeStruct((2 * TOKENS,), jnp.int32),
            jax.ShapeDtypeStruct((2 * TOKENS,), jnp.float32),
        ],
        mesh=plsc.VectorSubcoreMesh(core_axis_name="c", subcore_axis_name="s",
                                    num_cores=2, num_subcores=16),
        scratch_types=[
            pltpu.VMEM((2, 2, 16), jnp.int32),
            pltpu.VMEM((2, 2, 16), jnp.float32),
            pltpu.VMEM((CHUNK, HDIM), jnp.float32),
            pltpu.VMEM((CHUNK, HDIM), jnp.float32),
            pltpu.VMEM((16,), jnp.int32),
            pltpu.VMEM((16,), jnp.int32),
            pltpu.VMEM((16,), jnp.int32),
            pltpu.VMEM((16,), jnp.int32),
            pltpu.VMEM((2, 16), jnp.int32),
            pltpu.VMEM((2, 16), jnp.int32),
            pltpu.VMEM((2, 16), jnp.int32),
            pltpu.VMEM((2, 16), jnp.float32),
            pltpu.VMEM((2, 16), jnp.float32),
            pltpu.SemaphoreType.DMA,
            pltpu.SemaphoreType.DMA,
            pltpu.SemaphoreType.DMA,
            pltpu.SemaphoreType.DMA,
            pltpu.SemaphoreType.DMA,
            pltpu.SemaphoreType.DMA,
            pltpu.SemaphoreType.DMA,
            pltpu.SemaphoreType.DMA,
        ],
    )(_dispatch_body)


def kernel(x, W):
    p, counts = _router(x, W)
    pqi = p[:2].reshape(2, TOKENS // CHUNK, CHUNK).transpose(1, 0, 2)
    pqf = lax.bitcast_convert_type(
        p[2:4], jnp.float32).reshape(2, TOKENS // CHUNK, CHUNK).transpose(1, 0, 2)
    x_gathered, scatter_indices, scores_sorted = _make_dispatch()(x, pqi, pqf)
    num_tokens_per_expert = counts[0, :NUM_EXPERTS]
    return (x_gathered, num_tokens_per_expert, scatter_indices, scores_sorted)


# speedup vs baseline: 1.0773x; 1.0433x over previous
import functools

import jax
import jax.numpy as jnp
from jax import lax
from jax.experimental import pallas as pl
from jax.experimental.pallas import tpu as pltpu
from jax.experimental.pallas import tpu_sc as plsc

TOKENS = 8192
DIM = 4096
NUM_EXPERTS = 64
BT = 512
NB = TOKENS // BT
NW = 32
TPW = TOKENS // NW
CHUNK = 16
NCHUNK = TPW // CHUNK

_HI = jax.lax.Precision.HIGHEST


def _router_block(x_ref, w_ref, p_ref, counts_ref,
                  meta_s, probs_s, offs_s, carry_ref):
    b = pl.program_id(0)

    @pl.when(b == 0)
    def _init():
        carry_ref[...] = jnp.zeros_like(carry_ref)

    @pl.when(b < NB)
    def _phase_a():
        x = x_ref[...]
        w = w_ref[...]
        scores = lax.dot_general(x, w, (((1,), (1,)), ((), ())),
                                 preferred_element_type=jnp.float32)
        col = lax.broadcasted_iota(jnp.int32, (BT, NUM_EXPERTS), 1)
        m1 = jnp.max(scores, axis=1, keepdims=True)
        idx1 = jnp.min(jnp.where(scores == m1, col, NUM_EXPERTS), axis=1,
                       keepdims=True)
        is1 = col == idx1
        masked = jnp.where(is1, -jnp.inf, scores)
        m2 = jnp.max(masked, axis=1, keepdims=True)
        idx2 = jnp.min(jnp.where(masked == m2, col, NUM_EXPERTS), axis=1,
                       keepdims=True)
        is2 = col == idx2
        z = jnp.exp(m2 - m1)
        q1 = 1.0 / (1.0 + z)
        q2 = z / (1.0 + z)
        combined = is1.astype(jnp.float32) + is2.astype(jnp.float32)
        rowi = lax.broadcasted_iota(jnp.int32, (BT, BT), 0)
        coli = lax.broadcasted_iota(jnp.int32, (BT, BT), 1)
        tri = (rowi >= coli).astype(jnp.float32)
        incl = lax.dot_general(tri, combined, (((1,), (0,)), ((), ())),
                               preferred_element_type=jnp.float32)
        excl = incl - combined
        base = carry_ref[0:1, 0:NUM_EXPERTS]
        cnt = excl + base
        r0 = jnp.sum(jnp.where(is1, cnt, 0.0), axis=1, keepdims=True)
        r1 = jnp.sum(jnp.where(is2, cnt, 0.0), axis=1, keepdims=True)
        carry_ref[0:1, 0:NUM_EXPERTS] = base + incl[BT - 1:BT, :]
        zi = jnp.zeros((BT, 124), jnp.int32)
        meta_s[pl.ds(b * BT, BT), :] = jnp.concatenate(
            [idx1, idx2, r0.astype(jnp.int32), r1.astype(jnp.int32), zi],
            axis=1)
        zf = jnp.zeros((BT, 126), jnp.float32)
        probs_s[pl.ds(b * BT, BT), :] = jnp.concatenate([q1, q2, zf], axis=1)

        @pl.when(b == NB - 1)
        def _fin():
            counts_ref[...] = carry_ref[...]

    @pl.when(b >= NB)
    def _phase_b():
        @pl.when(b == NB)
        def _mkoffs():
            r128 = lax.broadcasted_iota(jnp.int32, (128, 128), 0)
            c128 = lax.broadcasted_iota(jnp.int32, (128, 128), 1)
            below = (r128 < c128).astype(jnp.float32)
            offs_s[...] = lax.dot_general(carry_ref[...], below,
                                          (((1,), (0,)), ((), ())),
                                          preferred_element_type=jnp.float32,
                                          precision=_HI)

        bb = b - NB
        ms = meta_s[pl.ds(bb * BT, BT), :]
        qs = probs_s[pl.ds(bb * BT, BT), :]
        col = lax.broadcasted_iota(jnp.int32, (BT, NUM_EXPERTS), 1)
        offs = offs_s[0:1, 0:NUM_EXPERTS]
        zero = jnp.zeros((BT, NUM_EXPERTS), jnp.float32)
        o0 = jnp.sum(jnp.where(col == ms[:, 0:1], offs + zero, 0.0),
                     axis=1, keepdims=True)
        o1 = jnp.sum(jnp.where(col == ms[:, 1:2], offs + zero, 0.0),
                     axis=1, keepdims=True)
        p0 = o0 + ms[:, 2:3].astype(jnp.float32)
        p1 = o1 + ms[:, 3:4].astype(jnp.float32)
        pt = jnp.transpose(jnp.concatenate([p0, p1], axis=1), (1, 0))
        qt = jnp.transpose(qs[:, 0:2], (1, 0))
        p_ref[...] = jnp.concatenate(
            [pt.astype(jnp.int32),
             lax.bitcast_convert_type(qt, jnp.int32),
             jnp.zeros((4, BT), jnp.int32)], axis=0)


_router = pl.pallas_call(
    _router_block,
    grid=(2 * NB,),
    in_specs=[
        pl.BlockSpec((BT, DIM), lambda b: (jnp.minimum(b, NB - 1), 0)),
        pl.BlockSpec((NUM_EXPERTS, DIM), lambda b: (0, 0)),
    ],
    out_specs=[
        pl.BlockSpec((8, BT), lambda b: (0, jnp.maximum(b - NB, 0))),
        pl.BlockSpec((8, 128), lambda b: (0, 0)),
    ],
    out_shape=[
        jax.ShapeDtypeStruct((8, TOKENS), jnp.int32),
        jax.ShapeDtypeStruct((8, 128), jnp.float32),
    ],
    scratch_shapes=[
        pltpu.VMEM((TOKENS, 128), jnp.int32),
        pltpu.VMEM((TOKENS, 128), jnp.float32),
        pltpu.VMEM((8, 128), jnp.float32),
        pltpu.VMEM((8, 128), jnp.float32),
    ],
)


HDIM = DIM // 2


def _dispatch_body(x_hbm, pqi_hbm, pqf_hbm, xg_hbm, sc_hbm, ss_hbm,
                   pqi_v, pqf_v, buf0, buf1, d00, d01, d10, d11,
                   p0w, p1w, vtw, q0w, q1w,
                   sem_l0, sem_l1, sem_s0, sem_s1, sem_w0, sem_w1,
                   sem_p0, sem_p1):
    cid = lax.axis_index("c")
    sid = lax.axis_index("s")
    wid = sid * 2 + cid
    base_tok = wid * TPW
    iota = lax.iota(jnp.int32, 16)

    def drain(src, dst, sem):
        pltpu.make_async_copy(src, dst, sem).wait()

    def load(tok, h, buf, sem):
        pltpu.async_copy(
            x_hbm.at[pl.ds(tok, CHUNK), pl.ds(h * HDIM, HDIM)], buf, sem)

    def drain_load(tok, h, buf, sem):
        pltpu.make_async_copy(
            x_hbm.at[pl.ds(tok, CHUNK), pl.ds(h * HDIM, HDIM)], buf,
            sem).wait()

    def pq_start(c, wi, sem):
        pltpu.async_copy(pqi_hbm.at[wid * NCHUNK + c], pqi_v.at[wi], sem)
        pltpu.async_copy(pqf_hbm.at[wid * NCHUNK + c], pqf_v.at[wi], sem)

    def pq_drain(c, wi, sem):
        pltpu.make_async_copy(
            pqi_hbm.at[wid * NCHUNK + c], pqi_v.at[wi], sem).wait()
        pltpu.make_async_copy(
            pqf_hbm.at[wid * NCHUNK + c], pqf_v.at[wi], sem).wait()

    load(base_tok, 0, buf0, sem_l0)
    pq_start(0, 0, sem_p0)

    @pl.loop(0, NCHUNK, step=2)
    def _outer(cbase):
        for cc in (0, 1):
            c = cbase + cc
            tok = base_tok + c * CHUNK
            wi = cc
            sem_w = sem_w0 if cc == 0 else sem_w1
            sem_p = sem_p0 if cc == 0 else sem_p1
            sem_pn = sem_p1 if cc == 0 else sem_p0
            pq_drain(c, wi, sem_p)
            @pl.when(c < NCHUNK - 1)
            def _pq_next():
                pq_start(c + 1, 1 - wi, sem_pn)
            p0v = pqi_v[wi, 0]
            p1v = pqi_v[wi, 1]
            drain_load(tok, 0, buf0, sem_l0)
            d00[...] = p0v
            d01[...] = p1v
            pltpu.async_copy(buf0, xg_hbm.at[d00, pl.ds(0, HDIM)], sem_s0)
            pltpu.async_copy(buf0, xg_hbm.at[d01, pl.ds(0, HDIM)], sem_s0)
            @pl.when(c >= 2)
            def _drain_words():
                drain(vtw.at[wi], sc_hbm.at[p0w.at[wi]], sem_w)
                drain(vtw.at[wi], sc_hbm.at[p1w.at[wi]], sem_w)
                drain(q0w.at[wi], ss_hbm.at[p0w.at[wi]], sem_w)
                drain(q1w.at[wi], ss_hbm.at[p1w.at[wi]], sem_w)
            p0w[wi, :] = p0v
            p1w[wi, :] = p1v
            vtw[wi, :] = tok + iota
            q0w[wi, :] = pqf_v[wi, 0]
            q1w[wi, :] = pqf_v[wi, 1]
            pltpu.async_copy(vtw.at[wi], sc_hbm.at[p0w.at[wi]], sem_w)
            pltpu.async_copy(vtw.at[wi], sc_hbm.at[p1w.at[wi]], sem_w)
            pltpu.async_copy(q0w.at[wi], ss_hbm.at[p0w.at[wi]], sem_w)
            pltpu.async_copy(q1w.at[wi], ss_hbm.at[p1w.at[wi]], sem_w)
            @pl.when(c >= 1)
            def _drain_s1():
                drain(buf1, xg_hbm.at[d10, pl.ds(HDIM, HDIM)], sem_s1)
                drain(buf1, xg_hbm.at[d11, pl.ds(HDIM, HDIM)], sem_s1)
            load(tok, 1, buf1, sem_l1)
            drain_load(tok, 1, buf1, sem_l1)
            d10[...] = p0v
            d11[...] = p1v
            pltpu.async_copy(buf1, xg_hbm.at[d10, pl.ds(HDIM, HDIM)], sem_s1)
            pltpu.async_copy(buf1, xg_hbm.at[d11, pl.ds(HDIM, HDIM)], sem_s1)
            drain(buf0, xg_hbm.at[d00, pl.ds(0, HDIM)], sem_s0)
            drain(buf0, xg_hbm.at[d01, pl.ds(0, HDIM)], sem_s0)
            @pl.when(c < NCHUNK - 1)
            def _next_load():
                load(tok + CHUNK, 0, buf0, sem_l0)

    drain(buf1, xg_hbm.at[d10, pl.ds(HDIM, HDIM)], sem_s1)
    drain(buf1, xg_hbm.at[d11, pl.ds(HDIM, HDIM)], sem_s1)
    for wi, sem_w in ((0, sem_w0), (1, sem_w1)):
        drain(vtw.at[wi], sc_hbm.at[p0w.at[wi]], sem_w)
        drain(vtw.at[wi], sc_hbm.at[p1w.at[wi]], sem_w)
        drain(q0w.at[wi], ss_hbm.at[p0w.at[wi]], sem_w)
        drain(q1w.at[wi], ss_hbm.at[p1w.at[wi]], sem_w)


@functools.cache
def _make_dispatch():
    return functools.partial(
        pl.kernel,
        out_type=[
            jax.ShapeDtypeStruct((2 * TOKENS, DIM), jnp.float32),
            jax.ShapeDtypeStruct((2 * TOKENS,), jnp.int32),
            jax.ShapeDtypeStruct((2 * TOKENS,), jnp.float32),
        ],
        mesh=plsc.VectorSubcoreMesh(core_axis_name="c", subcore_axis_name="s",
                                    num_cores=2, num_subcores=16),
        scratch_types=[
            pltpu.VMEM((2, 2, 16), jnp.int32),
            pltpu.VMEM((2, 2, 16), jnp.float32),
            pltpu.VMEM((CHUNK, HDIM), jnp.float32),
            pltpu.VMEM((CHUNK, HDIM), jnp.float32),
            pltpu.VMEM((16,), jnp.int32),
            pltpu.VMEM((16,), jnp.int32),
            pltpu.VMEM((16,), jnp.int32),
            pltpu.VMEM((16,), jnp.int32),
            pltpu.VMEM((2, 16), jnp.int32),
            pltpu.VMEM((2, 16), jnp.int32),
            pltpu.VMEM((2, 16), jnp.int32),
            pltpu.VMEM((2, 16), jnp.float32),
            pltpu.VMEM((2, 16), jnp.float32),
            pltpu.SemaphoreType.DMA,
            pltpu.SemaphoreType.DMA,
            pltpu.SemaphoreType.DMA,
            pltpu.SemaphoreType.DMA,
            pltpu.SemaphoreType.DMA,
            pltpu.SemaphoreType.DMA,
            pltpu.SemaphoreType.DMA,
            pltpu.SemaphoreType.DMA,
        ],
    )(_dispatch_body)


def kernel(x, W):
    p, counts = _router(x, W)
    pqi = p[:2].reshape(2, TOKENS // CHUNK, CHUNK).transpose(1, 0, 2)
    pqf = lax.bitcast_convert_type(
        p[2:4], jnp.float32).reshape(2, TOKENS // CHUNK, CHUNK).transpose(1, 0, 2)
    x_gathered, scatter_indices, scores_sorted = _make_dispatch()(x, pqi, pqf)
    num_tokens_per_expert = counts[0, :NUM_EXPERTS]
    return (x_gathered, num_tokens_per_expert, scatter_indices, scores_sorted)


# speedup vs baseline: 1.0858x; 1.0079x over previous
import functools

import jax
import jax.numpy as jnp
from jax import lax
from jax.experimental import pallas as pl
from jax.experimental.pallas import tpu as pltpu
from jax.experimental.pallas import tpu_sc as plsc

TOKENS = 8192
DIM = 4096
NUM_EXPERTS = 64
BT = 512
NB = TOKENS // BT
BT2 = 2048
NB2 = TOKENS // BT2
NW = 32
TPW = TOKENS // NW
CHUNK = 16
NCHUNK = TPW // CHUNK

_HI = jax.lax.Precision.HIGHEST


def _router_block(x_ref, w_ref, p_ref, counts_ref,
                  meta_s, probs_s, offs_s, carry_ref):
    b = pl.program_id(0)

    @pl.when(b == 0)
    def _init():
        carry_ref[...] = jnp.zeros_like(carry_ref)

    @pl.when(b < NB)
    def _phase_a():
        x = x_ref[...]
        w = w_ref[...]
        scores = lax.dot_general(x, w, (((1,), (1,)), ((), ())),
                                 preferred_element_type=jnp.float32)
        col = lax.broadcasted_iota(jnp.int32, (BT, NUM_EXPERTS), 1)
        m1 = jnp.max(scores, axis=1, keepdims=True)
        idx1 = jnp.min(jnp.where(scores == m1, col, NUM_EXPERTS), axis=1,
                       keepdims=True)
        is1 = col == idx1
        masked = jnp.where(is1, -jnp.inf, scores)
        m2 = jnp.max(masked, axis=1, keepdims=True)
        idx2 = jnp.min(jnp.where(masked == m2, col, NUM_EXPERTS), axis=1,
                       keepdims=True)
        is2 = col == idx2
        z = jnp.exp(m2 - m1)
        q1 = 1.0 / (1.0 + z)
        q2 = z / (1.0 + z)
        combined = is1.astype(jnp.float32) + is2.astype(jnp.float32)
        rowi = lax.broadcasted_iota(jnp.int32, (BT, BT), 0)
        coli = lax.broadcasted_iota(jnp.int32, (BT, BT), 1)
        tri = (rowi >= coli).astype(jnp.float32)
        incl = lax.dot_general(tri, combined, (((1,), (0,)), ((), ())),
                               preferred_element_type=jnp.float32)
        excl = incl - combined
        base = carry_ref[0:1, 0:NUM_EXPERTS]
        cnt = excl + base
        r0 = jnp.sum(jnp.where(is1, cnt, 0.0), axis=1, keepdims=True)
        r1 = jnp.sum(jnp.where(is2, cnt, 0.0), axis=1, keepdims=True)
        carry_ref[0:1, 0:NUM_EXPERTS] = base + incl[BT - 1:BT, :]
        zi = jnp.zeros((BT, 124), jnp.int32)
        meta_s[pl.ds(b * BT, BT), :] = jnp.concatenate(
            [idx1, idx2, r0.astype(jnp.int32), r1.astype(jnp.int32), zi],
            axis=1)
        zf = jnp.zeros((BT, 126), jnp.float32)
        probs_s[pl.ds(b * BT, BT), :] = jnp.concatenate([q1, q2, zf], axis=1)

        @pl.when(b == NB - 1)
        def _fin():
            counts_ref[...] = carry_ref[...]

    @pl.when(b >= NB)
    def _phase_b():
        @pl.when(b == NB)
        def _mkoffs():
            r128 = lax.broadcasted_iota(jnp.int32, (128, 128), 0)
            c128 = lax.broadcasted_iota(jnp.int32, (128, 128), 1)
            below = (r128 < c128).astype(jnp.float32)
            offs_s[...] = lax.dot_general(carry_ref[...], below,
                                          (((1,), (0,)), ((), ())),
                                          preferred_element_type=jnp.float32,
                                          precision=_HI)

        bb = b - NB
        ms = meta_s[pl.ds(bb * BT2, BT2), :]
        qs = probs_s[pl.ds(bb * BT2, BT2), :]
        col = lax.broadcasted_iota(jnp.int32, (BT2, NUM_EXPERTS), 1)
        offs = offs_s[0:1, 0:NUM_EXPERTS]
        zero = jnp.zeros((BT2, NUM_EXPERTS), jnp.float32)
        o0 = jnp.sum(jnp.where(col == ms[:, 0:1], offs + zero, 0.0),
                     axis=1, keepdims=True)
        o1 = jnp.sum(jnp.where(col == ms[:, 1:2], offs + zero, 0.0),
                     axis=1, keepdims=True)
        p0 = o0 + ms[:, 2:3].astype(jnp.float32)
        p1 = o1 + ms[:, 3:4].astype(jnp.float32)
        pt = jnp.transpose(jnp.concatenate([p0, p1], axis=1), (1, 0))
        qt = jnp.transpose(qs[:, 0:2], (1, 0))
        p_ref[...] = jnp.concatenate(
            [pt.astype(jnp.int32),
             lax.bitcast_convert_type(qt, jnp.int32),
             jnp.zeros((4, BT2), jnp.int32)], axis=0)


_router = pl.pallas_call(
    _router_block,
    grid=(NB + NB2,),
    in_specs=[
        pl.BlockSpec((BT, DIM), lambda b: (jnp.minimum(b, NB - 1), 0)),
        pl.BlockSpec((NUM_EXPERTS, DIM), lambda b: (0, 0)),
    ],
    out_specs=[
        pl.BlockSpec((8, BT2), lambda b: (0, jnp.maximum(b - NB, 0))),
        pl.BlockSpec((8, 128), lambda b: (0, 0)),
    ],
    out_shape=[
        jax.ShapeDtypeStruct((8, TOKENS), jnp.int32),
        jax.ShapeDtypeStruct((8, 128), jnp.float32),
    ],
    scratch_shapes=[
        pltpu.VMEM((TOKENS, 128), jnp.int32),
        pltpu.VMEM((TOKENS, 128), jnp.float32),
        pltpu.VMEM((8, 128), jnp.float32),
        pltpu.VMEM((8, 128), jnp.float32),
    ],
)


HDIM = DIM // 2


def _dispatch_body(x_hbm, pqi_hbm, pqf_hbm, xg_hbm, sc_hbm, ss_hbm,
                   pqi_v, pqf_v, buf0, buf1, d00, d01, d10, d11,
                   p0w, p1w, vtw, q0w, q1w,
                   sem_l0, sem_l1, sem_s0, sem_s1, sem_w0, sem_w1,
                   sem_p0, sem_p1):
    cid = lax.axis_index("c")
    sid = lax.axis_index("s")
    wid = sid * 2 + cid
    base_tok = wid * TPW
    iota = lax.iota(jnp.int32, 16)

    def drain(src, dst, sem):
        pltpu.make_async_copy(src, dst, sem).wait()

    def load(tok, h, buf, sem):
        pltpu.async_copy(
            x_hbm.at[pl.ds(tok, CHUNK), pl.ds(h * HDIM, HDIM)], buf, sem)

    def drain_load(tok, h, buf, sem):
        pltpu.make_async_copy(
            x_hbm.at[pl.ds(tok, CHUNK), pl.ds(h * HDIM, HDIM)], buf,
            sem).wait()

    def pq_start(c, wi, sem):
        pltpu.async_copy(pqi_hbm.at[wid * NCHUNK + c], pqi_v.at[wi], sem)
        pltpu.async_copy(pqf_hbm.at[wid * NCHUNK + c], pqf_v.at[wi], sem)

    def pq_drain(c, wi, sem):
        pltpu.make_async_copy(
            pqi_hbm.at[wid * NCHUNK + c], pqi_v.at[wi], sem).wait()
        pltpu.make_async_copy(
            pqf_hbm.at[wid * NCHUNK + c], pqf_v.at[wi], sem).wait()

    load(base_tok, 0, buf0, sem_l0)
    pq_start(0, 0, sem_p0)

    @pl.loop(0, NCHUNK, step=2)
    def _outer(cbase):
        for cc in (0, 1):
            c = cbase + cc
            tok = base_tok + c * CHUNK
            wi = cc
            sem_w = sem_w0 if cc == 0 else sem_w1
            sem_p = sem_p0 if cc == 0 else sem_p1
            sem_pn = sem_p1 if cc == 0 else sem_p0
            pq_drain(c, wi, sem_p)
            @pl.when(c < NCHUNK - 1)
            def _pq_next():
                pq_start(c + 1, 1 - wi, sem_pn)
            p0v = pqi_v[wi, 0]
            p1v = pqi_v[wi, 1]
            drain_load(tok, 0, buf0, sem_l0)
            d00[...] = p0v
            d01[...] = p1v
            pltpu.async_copy(buf0, xg_hbm.at[d00, pl.ds(0, HDIM)], sem_s0)
            pltpu.async_copy(buf0, xg_hbm.at[d01, pl.ds(0, HDIM)], sem_s0)
            @pl.when(c >= 2)
            def _drain_words():
                drain(vtw.at[wi], sc_hbm.at[p0w.at[wi]], sem_w)
                drain(vtw.at[wi], sc_hbm.at[p1w.at[wi]], sem_w)
                drain(q0w.at[wi], ss_hbm.at[p0w.at[wi]], sem_w)
                drain(q1w.at[wi], ss_hbm.at[p1w.at[wi]], sem_w)
            p0w[wi, :] = p0v
            p1w[wi, :] = p1v
            vtw[wi, :] = tok + iota
            q0w[wi, :] = pqf_v[wi, 0]
            q1w[wi, :] = pqf_v[wi, 1]
            pltpu.async_copy(vtw.at[wi], sc_hbm.at[p0w.at[wi]], sem_w)
            pltpu.async_copy(vtw.at[wi], sc_hbm.at[p1w.at[wi]], sem_w)
            pltpu.async_copy(q0w.at[wi], ss_hbm.at[p0w.at[wi]], sem_w)
            pltpu.async_copy(q1w.at[wi], ss_hbm.at[p1w.at[wi]], sem_w)
            @pl.when(c >= 1)
            def _drain_s1():
                drain(buf1, xg_hbm.at[d10, pl.ds(HDIM, HDIM)], sem_s1)
                drain(buf1, xg_hbm.at[d11, pl.ds(HDIM, HDIM)], sem_s1)
            load(tok, 1, buf1, sem_l1)
            drain_load(tok, 1, buf1, sem_l1)
            d10[...] = p0v
            d11[...] = p1v
            pltpu.async_copy(buf1, xg_hbm.at[d10, pl.ds(HDIM, HDIM)], sem_s1)
            pltpu.async_copy(buf1, xg_hbm.at[d11, pl.ds(HDIM, HDIM)], sem_s1)
            drain(buf0, xg_hbm.at[d00, pl.ds(0, HDIM)], sem_s0)
            drain(buf0, xg_hbm.at[d01, pl.ds(0, HDIM)], sem_s0)
            @pl.when(c < NCHUNK - 1)
            def _next_load():
                load(tok + CHUNK, 0, buf0, sem_l0)

    drain(buf1, xg_hbm.at[d10, pl.ds(HDIM, HDIM)], sem_s1)
    drain(buf1, xg_hbm.at[d11, pl.ds(HDIM, HDIM)], sem_s1)
    for wi, sem_w in ((0, sem_w0), (1, sem_w1)):
        drain(vtw.at[wi], sc_hbm.at[p0w.at[wi]], sem_w)
        drain(vtw.at[wi], sc_hbm.at[p1w.at[wi]], sem_w)
        drain(q0w.at[wi], ss_hbm.at[p0w.at[wi]], sem_w)
        drain(q1w.at[wi], ss_hbm.at[p1w.at[wi]], sem_w)


@functools.cache
def _make_dispatch():
    return functools.partial(
        pl.kernel,
        out_type=[
            jax.ShapeDtypeStruct((2 * TOKENS, DIM), jnp.float32),
            jax.ShapeDtypeStruct((2 * TOKENS,), jnp.int32),
            jax.ShapeDtypeStruct((2 * TOKENS,), jnp.float32),
        ],
        mesh=plsc.VectorSubcoreMesh(core_axis_name="c", subcore_axis_name="s",
                                    num_cores=2, num_subcores=16),
        scratch_types=[
            pltpu.VMEM((2, 2, 16), jnp.int32),
            pltpu.VMEM((2, 2, 16), jnp.float32),
            pltpu.VMEM((CHUNK, HDIM), jnp.float32),
            pltpu.VMEM((CHUNK, HDIM), jnp.float32),
            pltpu.VMEM((16,), jnp.int32),
            pltpu.VMEM((16,), jnp.int32),
            pltpu.VMEM((16,), jnp.int32),
            pltpu.VMEM((16,), jnp.int32),
            pltpu.VMEM((2, 16), jnp.int32),
            pltpu.VMEM((2, 16), jnp.int32),
            pltpu.VMEM((2, 16), jnp.int32),
            pltpu.VMEM((2, 16), jnp.float32),
            pltpu.VMEM((2, 16), jnp.float32),
            pltpu.SemaphoreType.DMA,
            pltpu.SemaphoreType.DMA,
            pltpu.SemaphoreType.DMA,
            pltpu.SemaphoreType.DMA,
            pltpu.SemaphoreType.DMA,
            pltpu.SemaphoreType.DMA,
            pltpu.SemaphoreType.DMA,
            pltpu.SemaphoreType.DMA,
        ],
    )(_dispatch_body)


def kernel(x, W):
    p, counts = _router(x, W)
    pqi = p[:2].reshape(2, TOKENS // CHUNK, CHUNK).transpose(1, 0, 2)
    pqf = lax.bitcast_convert_type(
        p[2:4], jnp.float32).reshape(2, TOKENS // CHUNK, CHUNK).transpose(1, 0, 2)
    x_gathered, scatter_indices, scores_sorted = _make_dispatch()(x, pqi, pqf)
    num_tokens_per_expert = counts[0, :NUM_EXPERTS]
    return (x_gathered, num_tokens_per_expert, scatter_indices, scores_sorted)
